# Initial kernel scaffold; baseline (speedup 1.0000x reference)
import functools

import jax
import jax.numpy as jnp
from jax import lax
from jax.experimental import pallas as pl
from jax.experimental.pallas import tpu as pltpu
from jax.experimental.pallas import tpu_sc as plsc

NC = 2
NS = 16
NW = NC * NS
B = 128

_MESH = dict(core_axis_name="c", subcore_axis_name="s", num_cores=NC,
             num_subcores=NS)


def _deg_kernel(npad, cpt):
    stripe = npad // NS

    @functools.partial(
        pl.kernel,
        out_type=jax.ShapeDtypeStruct((NC, npad, 16), jnp.float32),
        mesh=plsc.VectorSubcoreMesh(**_MESH),
        scratch_types=[
            pltpu.VMEM((cpt, B), jnp.int32),
            pltpu.VMEM((B, 16), jnp.float32),
        ],
    )
    def deg(cols_hbm, zeros_hbm, ones_hbm, out_hbm, colsv, onesv):
        c = lax.axis_index("c")
        s = lax.axis_index("s")
        wid = s * NC + c
        base = s * stripe
        pltpu.sync_copy(cols_hbm.at[wid], colsv)
        pltpu.sync_copy(ones_hbm, onesv)

        def zeroed(deg_sp):
            pltpu.sync_copy(zeros_hbm.at[pl.ds(base, stripe)],
                            deg_sp.at[pl.ds(base, stripe)])
            plsc.subcore_barrier()

            def step(j, carry):
                pltpu.sync_copy(onesv, deg_sp.at[colsv.at[j]], add=True)
                return carry
            lax.fori_loop(0, cpt, step, 0)
            plsc.subcore_barrier()
            pltpu.sync_copy(deg_sp.at[pl.ds(base, stripe)],
                            out_hbm.at[c, pl.ds(base, stripe)])
        pl.run_scoped(zeroed, pltpu.VMEM_SHARED((npad, 16), jnp.float32))

    return deg


def _agg_kernel(npad, cpt, d):
    stripe = npad // NS

    @functools.partial(
        pl.kernel,
        out_type=jax.ShapeDtypeStruct((NC, npad, d), jnp.float32),
        mesh=plsc.VectorSubcoreMesh(**_MESH),
        scratch_types=[
            pltpu.VMEM((cpt, B), jnp.int32),
            pltpu.VMEM((cpt, B), jnp.int32),
            pltpu.VMEM((B, d), jnp.float32),
        ],
    )
    def agg(h_hbm, rows_hbm, cols_hbm, zeros_hbm, out_hbm, rowsv, colsv, gbuf):
        c = lax.axis_index("c")
        s = lax.axis_index("s")
        wid = s * NC + c
        base = s * stripe
        pltpu.sync_copy(rows_hbm.at[wid], rowsv)
        pltpu.sync_copy(cols_hbm.at[wid], colsv)

        def scoped(acc_sp):
            @pl.when(c == 0)
            def _():
                pltpu.sync_copy(h_hbm.at[pl.ds(base, stripe)],
                                acc_sp.at[pl.ds(base, stripe)])

            @pl.when(c == 1)
            def _():
                pltpu.sync_copy(zeros_hbm.at[pl.ds(base, stripe)],
                                acc_sp.at[pl.ds(base, stripe)])
            plsc.subcore_barrier()

            def step(j, carry):
                pltpu.sync_copy(h_hbm.at[rowsv.at[j]], gbuf)
                pltpu.sync_copy(gbuf, acc_sp.at[colsv.at[j]], add=True)
                return carry
            lax.fori_loop(0, cpt, step, 0)
            plsc.subcore_barrier()
            pltpu.sync_copy(acc_sp.at[pl.ds(base, stripe)],
                            out_hbm.at[c, pl.ds(base, stripe)])
        pl.run_scoped(scoped, pltpu.VMEM_SHARED((npad, d), jnp.float32))

    return agg


def _tc_first(degp, x, w0, n, npad, d):
    def body(degp_ref, x_ref, w_ref, dinv_ref, h_ref):
        cnt = (jnp.sum(degp_ref[0], axis=1, keepdims=True)
               + jnp.sum(degp_ref[1], axis=1, keepdims=True)) * (1.0 / 16.0)
        dinv = lax.rsqrt(cnt + 1.0)
        dinv_ref[...] = dinv
        xw = jnp.dot(x_ref[...], w_ref[...],
                     preferred_element_type=jnp.float32)
        h_ref[0:n, :] = dinv[0:n] * xw
        h_ref[n:npad, :] = jnp.zeros((npad - n, d), jnp.float32)

    return pl.pallas_call(
        body,
        out_shape=[jax.ShapeDtypeStruct((npad, 1), jnp.float32),
                   jax.ShapeDtypeStruct((npad, d), jnp.float32)],
    )(degp, x, w0)


def _tc_mid(p, dinv, bias, w, npad, d):
    def body(p_ref, dinv_ref, b_ref, w_ref, h_ref):
        dinv = dinv_ref[...]
        g = jnp.maximum(dinv * (p_ref[0] + p_ref[1]) + b_ref[...], 0.0)
        h_ref[...] = dinv * jnp.dot(g, w_ref[...],
                                    preferred_element_type=jnp.float32)

    return pl.pallas_call(
        body,
        out_shape=jax.ShapeDtypeStruct((npad, d), jnp.float32),
    )(p, dinv, bias, w)


def _tc_final(p, dinv, bias, npad, d):
    def body(p_ref, dinv_ref, b_ref, o_ref):
        o_ref[...] = dinv_ref[...] * (p_ref[0] + p_ref[1]) + b_ref[...]

    return pl.pallas_call(
        body,
        out_shape=jax.ShapeDtypeStruct((npad, d), jnp.float32),
    )(p, dinv, bias)


def kernel(x, edge_index, edge_weight, W0, b0, W1, b1, W2, b2):
    n, d = x.shape
    e = edge_index.shape[1]
    del edge_weight

    npad = (n // B + 1) * B
    epad = pl.cdiv(e, NW * B) * NW * B
    cpt = epad // (NW * B)
    pad = epad - e

    ar = jnp.arange(pad, dtype=jnp.int32)
    rows = jnp.concatenate([edge_index[0], ar % n]).reshape(NW, cpt, B)
    cols = jnp.concatenate(
        [edge_index[1], n + ar % (npad - n)]).reshape(NW, cpt, B)

    zeros_nd = jnp.zeros((npad, d), jnp.float32)
    zeros_16 = jnp.zeros((npad, 16), jnp.float32)
    ones_16 = jnp.ones((B, 16), jnp.float32)

    degp = _deg_kernel(npad, cpt)(cols, zeros_16, ones_16)
    dinv, h = _tc_first(degp, x, W0, n, npad, d)

    agg = _agg_kernel(npad, cpt, d)
    p = agg(h, rows, cols, zeros_nd)
    h = _tc_mid(p, dinv, jnp.reshape(b0, (1, d)), W1, npad, d)
    p = agg(h, rows, cols, zeros_nd)
    h = _tc_mid(p, dinv, jnp.reshape(b1, (1, d)), W2, npad, d)
    p = agg(h, rows, cols, zeros_nd)
    logits = _tc_final(p, dinv, jnp.reshape(b2, (1, d)), npad, d)
    return logits[:n]


# speedup vs baseline: 19.1445x; 19.1445x over previous
import functools

import jax
import jax.numpy as jnp
from jax import lax
from jax.experimental import pallas as pl
from jax.experimental.pallas import tpu as pltpu
from jax.experimental.pallas import tpu_sc as plsc

NC = 2
NS = 16
NW = NC * NS
B = 128

_MESH = dict(core_axis_name="c", subcore_axis_name="s", num_cores=NC,
             num_subcores=NS)


def _deg_kernel(npad, cpt):
    stripe = npad // NS

    @functools.partial(
        pl.kernel,
        out_type=jax.ShapeDtypeStruct((NC, npad, 16), jnp.float32),
        mesh=plsc.VectorSubcoreMesh(**_MESH),
        scratch_types=[
            pltpu.VMEM((cpt, B), jnp.int32),
            pltpu.VMEM((B, 16), jnp.float32),
            pltpu.VMEM_SHARED((npad, 16), jnp.float32),
        ],
    )
    def deg(cols_hbm, zeros_hbm, ones_hbm, out_hbm, colsv, onesv, deg_sp):
        c = lax.axis_index("c")
        s = lax.axis_index("s")
        wid = s * NC + c
        base = s * stripe
        pltpu.sync_copy(cols_hbm.at[wid], colsv)
        pltpu.sync_copy(ones_hbm, onesv)
        pltpu.sync_copy(zeros_hbm.at[pl.ds(base, stripe)],
                        deg_sp.at[pl.ds(base, stripe)])
        plsc.subcore_barrier()

        def step(j, carry):
            pltpu.sync_copy(onesv, deg_sp.at[colsv.at[j]], add=True)
            return carry
        lax.fori_loop(0, cpt, step, 0)
        plsc.subcore_barrier()
        pltpu.sync_copy(deg_sp.at[pl.ds(base, stripe)],
                        out_hbm.at[c, pl.ds(base, stripe)])

    return deg


def _agg_kernel(npad, cpt, d):
    stripe = npad // NS

    @functools.partial(
        pl.kernel,
        out_type=jax.ShapeDtypeStruct((NC, npad, d), jnp.float32),
        mesh=plsc.VectorSubcoreMesh(**_MESH),
        scratch_types=[
            pltpu.VMEM((cpt, B), jnp.int32),
            pltpu.VMEM((cpt, B), jnp.int32),
            pltpu.VMEM((B, d), jnp.float32),
            pltpu.VMEM_SHARED((npad, d), jnp.float32),
        ],
    )
    def agg(h_hbm, rows_hbm, cols_hbm, zeros_hbm, out_hbm,
            rowsv, colsv, gbuf, acc_sp):
        c = lax.axis_index("c")
        s = lax.axis_index("s")
        wid = s * NC + c
        base = s * stripe
        pltpu.sync_copy(rows_hbm.at[wid], rowsv)
        pltpu.sync_copy(cols_hbm.at[wid], colsv)

        @pl.when(c == 0)
        def _():
            pltpu.sync_copy(h_hbm.at[pl.ds(base, stripe)],
                            acc_sp.at[pl.ds(base, stripe)])

        @pl.when(c == 1)
        def _():
            pltpu.sync_copy(zeros_hbm.at[pl.ds(base, stripe)],
                            acc_sp.at[pl.ds(base, stripe)])
        plsc.subcore_barrier()

        def step(j, carry):
            pltpu.sync_copy(h_hbm.at[rowsv.at[j]], gbuf)
            pltpu.sync_copy(gbuf, acc_sp.at[colsv.at[j]], add=True)
            return carry
        lax.fori_loop(0, cpt, step, 0)
        plsc.subcore_barrier()
        pltpu.sync_copy(acc_sp.at[pl.ds(base, stripe)],
                        out_hbm.at[c, pl.ds(base, stripe)])

    return agg


def _tc_first(degp, x, w0, n, npad, d):
    def body(degp_ref, x_ref, w_ref, dinv_ref, h_ref):
        cnt = (jnp.sum(degp_ref[0], axis=1, keepdims=True)
               + jnp.sum(degp_ref[1], axis=1, keepdims=True)) * (1.0 / 16.0)
        dinv = lax.rsqrt(cnt + 1.0)
        dinv_ref[...] = dinv
        xw = jnp.dot(x_ref[...], w_ref[...],
                     preferred_element_type=jnp.float32)
        h_ref[0:n, :] = dinv[0:n] * xw
        h_ref[n:npad, :] = jnp.zeros((npad - n, d), jnp.float32)

    return pl.pallas_call(
        body,
        out_shape=[jax.ShapeDtypeStruct((npad, 1), jnp.float32),
                   jax.ShapeDtypeStruct((npad, d), jnp.float32)],
    )(degp, x, w0)


def _tc_mid(p, dinv, bias, w, npad, d):
    def body(p_ref, dinv_ref, b_ref, w_ref, h_ref):
        dinv = dinv_ref[...]
        g = jnp.maximum(dinv * (p_ref[0] + p_ref[1]) + b_ref[...], 0.0)
        h_ref[...] = dinv * jnp.dot(g, w_ref[...],
                                    preferred_element_type=jnp.float32)

    return pl.pallas_call(
        body,
        out_shape=jax.ShapeDtypeStruct((npad, d), jnp.float32),
    )(p, dinv, bias, w)


def _tc_final(p, dinv, bias, npad, d):
    def body(p_ref, dinv_ref, b_ref, o_ref):
        o_ref[...] = dinv_ref[...] * (p_ref[0] + p_ref[1]) + b_ref[...]

    return pl.pallas_call(
        body,
        out_shape=jax.ShapeDtypeStruct((npad, d), jnp.float32),
    )(p, dinv, bias)


def kernel(x, edge_index, edge_weight, W0, b0, W1, b1, W2, b2):
    n, d = x.shape
    e = edge_index.shape[1]
    del edge_weight

    npad = (n // B + 1) * B
    epad = pl.cdiv(e, NW * B) * NW * B
    cpt = epad // (NW * B)
    pad = epad - e

    ar = jnp.arange(pad, dtype=jnp.int32)
    rows = jnp.concatenate([edge_index[0], ar % n]).reshape(NW, cpt, B)
    cols = jnp.concatenate(
        [edge_index[1], n + ar % (npad - n)]).reshape(NW, cpt, B)

    zeros_nd = jnp.zeros((npad, d), jnp.float32)
    zeros_16 = jnp.zeros((npad, 16), jnp.float32)
    ones_16 = jnp.ones((B, 16), jnp.float32)

    degp = _deg_kernel(npad, cpt)(cols, zeros_16, ones_16)
    dinv, h = _tc_first(degp, x, W0, n, npad, d)

    agg = _agg_kernel(npad, cpt, d)
    p = agg(h, rows, cols, zeros_nd)
    h = _tc_mid(p, dinv, jnp.reshape(b0, (1, d)), W1, npad, d)
    p = agg(h, rows, cols, zeros_nd)
    h = _tc_mid(p, dinv, jnp.reshape(b1, (1, d)), W2, npad, d)
    p = agg(h, rows, cols, zeros_nd)
    logits = _tc_final(p, dinv, jnp.reshape(b2, (1, d)), npad, d)
    return logits[:n]


# speedup vs baseline: 27.2268x; 1.4222x over previous
import functools

import jax
import jax.numpy as jnp
from jax import lax
from jax.experimental import pallas as pl
from jax.experimental.pallas import tpu as pltpu
from jax.experimental.pallas import tpu_sc as plsc

NC = 2
NS = 16
NW = NC * NS
B = 128

_MESH = dict(core_axis_name="c", subcore_axis_name="s", num_cores=NC,
             num_subcores=NS)


def _deg_kernel(npad, cpt):
    stripe = npad // NS

    @functools.partial(
        pl.kernel,
        out_type=jax.ShapeDtypeStruct((NC, npad, 16), jnp.float32),
        mesh=plsc.VectorSubcoreMesh(**_MESH),
        scratch_types=[
            pltpu.VMEM((cpt, B), jnp.int32),
            pltpu.VMEM((B, 16), jnp.float32),
            pltpu.VMEM_SHARED((npad, 16), jnp.float32),
        ],
    )
    def deg(cols_hbm, zeros_hbm, ones_hbm, out_hbm, colsv, onesv, deg_sp):
        c = lax.axis_index("c")
        s = lax.axis_index("s")
        wid = s * NC + c
        base = s * stripe
        pltpu.sync_copy(cols_hbm.at[wid], colsv)
        pltpu.sync_copy(ones_hbm, onesv)
        pltpu.sync_copy(zeros_hbm.at[pl.ds(base, stripe)],
                        deg_sp.at[pl.ds(base, stripe)])
        plsc.subcore_barrier()

        def step(j, carry):
            pltpu.sync_copy(onesv, deg_sp.at[colsv.at[j]], add=True)
            return carry
        lax.fori_loop(0, cpt, step, 0)
        plsc.subcore_barrier()
        pltpu.sync_copy(deg_sp.at[pl.ds(base, stripe)],
                        out_hbm.at[c, pl.ds(base, stripe)])

    return deg


def _agg_kernel(npad, cpt, d):
    stripe = npad // NS

    @functools.partial(
        pl.kernel,
        out_type=jax.ShapeDtypeStruct((NC, npad, d), jnp.float32),
        mesh=plsc.VectorSubcoreMesh(**_MESH),
        scratch_types=[
            pltpu.VMEM_SHARED((npad, d), jnp.float32),
            pltpu.SemaphoreType.DMA,
            pltpu.SemaphoreType.DMA,
        ],
    )
    def agg(h_hbm, rows_hbm, cols_hbm, zeros_hbm, out_hbm,
            acc_sp, sem0, sem1):
        c = lax.axis_index("c")
        s = lax.axis_index("s")
        wid = s * NC + c
        base = s * stripe
        half = (cpt + 1) // 2
        spans = []
        j0 = 0
        while j0 < cpt:
            spans.append((j0, min(half, cpt - j0)))
            j0 += half

        def scoped(rowsv, colsv, gbuf0, gbuf1):
            sub = stripe // 8

            def init(k, carry):
                off = base + k * sub

                @pl.when(c == 0)
                def _():
                    pltpu.sync_copy(h_hbm.at[pl.ds(off, sub)],
                                    acc_sp.at[pl.ds(off, sub)])

                @pl.when(c == 1)
                def _():
                    pltpu.sync_copy(zeros_hbm.at[pl.ds(off, sub)],
                                    acc_sp.at[pl.ds(off, sub)])
                return carry
            lax.fori_loop(0, 8, init, 0, unroll=False)
            plsc.subcore_barrier()

            def gather(jl, buf, sem):
                pltpu.async_copy(h_hbm.at[rowsv.at[jl]], buf, sem)

            def wait_scatter(jl, buf, sem):
                pltpu.make_async_copy(h_hbm.at[rowsv.at[jl]], buf,
                                      sem).wait()
                pltpu.sync_copy(buf, acc_sp.at[colsv.at[jl]], add=True)

            for p0, cnt in spans:
                pltpu.sync_copy(rows_hbm.at[wid, pl.ds(p0, cnt)],
                                rowsv.at[pl.ds(0, cnt)])
                pltpu.sync_copy(cols_hbm.at[wid, pl.ds(p0, cnt)],
                                colsv.at[pl.ds(0, cnt)])
                gather(0, gbuf0, sem0)

                def step(i, carry):
                    gather(2 * i + 1, gbuf1, sem1)
                    wait_scatter(2 * i, gbuf0, sem0)
                    gather(2 * i + 2, gbuf0, sem0)
                    wait_scatter(2 * i + 1, gbuf1, sem1)
                    return carry
                lax.fori_loop(0, (cnt - 1) // 2, step, 0)
                if cnt % 2 == 1:
                    wait_scatter(cnt - 1, gbuf0, sem0)
                else:
                    gather(cnt - 1, gbuf1, sem1)
                    wait_scatter(cnt - 2, gbuf0, sem0)
                    wait_scatter(cnt - 1, gbuf1, sem1)
            plsc.subcore_barrier()

            def drain(k, carry):
                off = base + k * sub
                pltpu.sync_copy(acc_sp.at[pl.ds(off, sub)],
                                out_hbm.at[c, pl.ds(off, sub)])
                return carry
            lax.fori_loop(0, 8, drain, 0, unroll=False)
        pl.run_scoped(scoped,
                      pltpu.VMEM(((cpt + 1) // 2, B), jnp.int32),
                      pltpu.VMEM(((cpt + 1) // 2, B), jnp.int32),
                      pltpu.VMEM((B, d), jnp.float32),
                      pltpu.VMEM((B, d), jnp.float32))

    return agg


def _tc_first(degp, x, w0, n, npad, d):
    def body(degp_ref, x_ref, w_ref, dinv_ref, h_ref):
        cnt = (jnp.sum(degp_ref[0], axis=1, keepdims=True)
               + jnp.sum(degp_ref[1], axis=1, keepdims=True)) * (1.0 / 16.0)
        dinv = lax.rsqrt(cnt + 1.0)
        dinv_ref[...] = dinv
        xw = jnp.dot(x_ref[...], w_ref[...],
                     preferred_element_type=jnp.float32)
        h_ref[0:n, :] = dinv[0:n] * xw
        h_ref[n:npad, :] = jnp.zeros((npad - n, d), jnp.float32)

    return pl.pallas_call(
        body,
        out_shape=[jax.ShapeDtypeStruct((npad, 1), jnp.float32),
                   jax.ShapeDtypeStruct((npad, d), jnp.float32)],
    )(degp, x, w0)


def _tc_mid(p, dinv, bias, w, npad, d):
    def body(p_ref, dinv_ref, b_ref, w_ref, h_ref):
        dinv = dinv_ref[...]
        g = jnp.maximum(dinv * (p_ref[0] + p_ref[1]) + b_ref[...], 0.0)
        h_ref[...] = dinv * jnp.dot(g, w_ref[...],
                                    preferred_element_type=jnp.float32)

    return pl.pallas_call(
        body,
        out_shape=jax.ShapeDtypeStruct((npad, d), jnp.float32),
    )(p, dinv, bias, w)


def _tc_final(p, dinv, bias, npad, d):
    def body(p_ref, dinv_ref, b_ref, o_ref):
        o_ref[...] = dinv_ref[...] * (p_ref[0] + p_ref[1]) + b_ref[...]

    return pl.pallas_call(
        body,
        out_shape=jax.ShapeDtypeStruct((npad, d), jnp.float32),
    )(p, dinv, bias)


def kernel(x, edge_index, edge_weight, W0, b0, W1, b1, W2, b2):
    n, d = x.shape
    e = edge_index.shape[1]
    del edge_weight

    npad = (n // (NS * B) + 1) * NS * B
    epad = pl.cdiv(e, NW * B) * NW * B
    cpt = epad // (NW * B)
    pad = epad - e

    ar = jnp.arange(pad, dtype=jnp.int32)
    rows = jnp.concatenate([edge_index[0], ar % n]).reshape(NW, cpt, B)
    cols = jnp.concatenate(
        [edge_index[1], n + ar % (npad - n)]).reshape(NW, cpt, B)

    zeros_nd = jnp.zeros((npad, d), jnp.float32)
    zeros_16 = jnp.zeros((npad, 16), jnp.float32)
    ones_16 = jnp.ones((B, 16), jnp.float32)

    degp = _deg_kernel(npad, cpt)(cols, zeros_16, ones_16)
    dinv, h = _tc_first(degp, x, W0, n, npad, d)

    agg = _agg_kernel(npad, cpt, d)
    p = agg(h, rows, cols, zeros_nd)
    h = _tc_mid(p, dinv, jnp.reshape(b0, (1, d)), W1, npad, d)
    p = agg(h, rows, cols, zeros_nd)
    h = _tc_mid(p, dinv, jnp.reshape(b1, (1, d)), W2, npad, d)
    p = agg(h, rows, cols, zeros_nd)
    logits = _tc_final(p, dinv, jnp.reshape(b2, (1, d)), npad, d)
    return logits[:n]
